# baseline (speedup 1.0000x reference)
import jax
import jax.numpy as jnp
import numpy as np
from jax.experimental import pallas as pl
from jax.experimental.pallas import tpu as pltpu

PARTITIONS = 8
D_H = 128
D = 128
OBS = 8
B = 1024
NEI = 64

BB = 16


def _transpose_kernel(x_ref, o_ref):
    o_ref[...] = jnp.transpose(x_ref[...], (0, 2, 1))


def _fused_kernel(x_ego_ref, x_nei16_ref, x_nei_T_ref, W_big_ref, b_te_ref,
                  W1_ref, b1_ref, W2_ref, b2_ref, W3_ref, b3_ref,
                  W_ce_ref, b_ce_ref, f_re_ref, re_ref):
    bb = x_ego_ref.shape[0]
    f32 = jnp.float32
    x_ego = x_ego_ref[...]
    ego_last = x_ego[:, OBS - 1:OBS, :]

    x16 = x_nei16_ref[...]
    xw = jnp.dot(x16.reshape(bb * NEI, OBS * 2), W_big_ref[...],
                 preferred_element_type=f32).reshape(bb, NEI, OBS, D_H)
    f_nei = jax.nn.relu(xw - xw[:, :, OBS - 1:OBS, :] + b_te_ref[...])

    xe = jnp.dot(x_ego.reshape(bb, OBS * 2), W_big_ref[...],
                 preferred_element_type=f32).reshape(bb, OBS, D_H)
    f_ego = jax.nn.relu(xe - xe[:, OBS - 1:OBS, :] + b_te_ref[...])
    f_ego = f_ego.reshape(bb, 1, OBS, D_H)

    f = (f_ego * f_nei).reshape(bb * NEI * OBS, D_H)

    h = jax.nn.relu(
        jnp.dot(f, W1_ref[...], preferred_element_type=f32) + b1_ref[...])
    h = jax.nn.relu(
        jnp.dot(h, W2_ref[...], preferred_element_type=f32) + b2_ref[...])
    h4 = h.reshape(bb, NEI, OBS, D_H)
    f_ts = [
        jax.nn.relu(
            jnp.dot(h4[:, :, t, :].reshape(bb * NEI, D_H), W3_ref[...],
                    preferred_element_type=f32) + b3_ref[...])
        for t in range(OBS)
    ]
    f_wide = jnp.concatenate(f_ts, axis=-1).reshape(bb, NEI, OBS * (D // 2))
    f_re_ref[...] = f_wide

    xT = x_nei_T_ref[...].reshape(bb, OBS, 2, NEI)
    ego_dx = (ego_last[:, :, 0] - x_ego[:, :, 0])[..., None]
    ego_dy = (ego_last[:, :, 1] - x_ego[:, :, 1])[..., None]
    px_t = xT[:, :, 0, :] + ego_dx
    py_t = xT[:, :, 1, :] + ego_dy
    dist_t = jnp.sqrt(px_t * px_t + py_t * py_t)
    ang_t = jnp.arctan2(px_t, py_t) % (2.0 * np.pi)
    idx_t = (ang_t / (2.0 * np.pi / PARTITIONS)).astype(jnp.int32)
    valid = (jnp.abs(px_t + py_t) > 1e-6) & (dist_t > 0.005)
    t_iota = jax.lax.broadcasted_iota(jnp.int32, (1, OBS, 1), 1)
    code_t = jnp.where(valid, idx_t + PARTITIONS * t_iota, -1)

    c_iota = jax.lax.broadcasted_iota(jnp.int32, (1, 1, 1, OBS * PARTITIONS), 3)

    tp_iota = jax.lax.broadcasted_iota(jnp.int32, (1, OBS, PARTITIONS, 1), 1)
    p_iota = jax.lax.broadcasted_iota(jnp.int32, (1, OBS, PARTITIONS, 1), 2)
    M2 = (code_t[:, :, None, :] == PARTITIONS * tp_iota + p_iota)
    M2 = M2.astype(f32).reshape(bb, OBS * PARTITIONS, NEI)
    out2 = jax.lax.dot_general(
        M2, f_wide, (((2,), (1,)), ((0,), (0,))),
        preferred_element_type=f32)
    r_iota = jax.lax.broadcasted_iota(jnp.int32, (1, OBS * PARTITIONS, 1), 1)
    l_iota = jax.lax.broadcasted_iota(
        jnp.int32, (1, 1, OBS * (D // 2)), 2)
    Dm = (r_iota // PARTITIONS == l_iota // (D // 2)).astype(f32)
    masked = out2 * Dm
    rr = jax.lax.broadcasted_iota(jnp.int32, (OBS * (D // 2), 1), 0)
    rc = jax.lax.broadcasted_iota(jnp.int32, (1, D // 2), 1)
    R = (rr % (D // 2) == rc).astype(f32)
    out_re = jax.lax.dot_general(
        masked, R, (((2,), (0,)), ((), ())),
        preferred_element_type=f32)

    M_t = (code_t[..., None] == c_iota).astype(f32)
    ones = jnp.ones((bb, 1, OBS, NEI), f32)
    V = jnp.concatenate([dist_t[:, None], ang_t[:, None], ones], axis=1)
    out_geo = jax.lax.dot_general(
        V.reshape(bb, 3, OBS * NEI),
        M_t.reshape(bb, OBS * NEI, OBS * PARTITIONS),
        (((2,), (1,)), ((0,), (0,))),
        preferred_element_type=f32)

    dsum = jnp.transpose(out_geo[:, 0:1, :], (0, 2, 1))
    asum = jnp.transpose(out_geo[:, 1:2, :], (0, 2, 1))
    cnt = jnp.transpose(out_geo[:, 2:3, :], (0, 2, 1))

    inv_n = 1.0 / (cnt + 0.0001)
    re_part = out_re * inv_n
    d_mean = dsum * inv_n
    a_mean = asum * inv_n

    wc0 = W_ce_ref[0:1, :].reshape(1, 1, D // 2)
    wc1 = W_ce_ref[1:2, :].reshape(1, 1, D // 2)
    f_pos = jax.nn.relu(d_mean * wc0 + a_mean * wc1
                        + b_ce_ref[...].reshape(1, 1, D // 2))

    re_ref[...] = jnp.concatenate([re_part, f_pos], axis=-1)


@jax.jit
def kernel(x_ego_2d, x_nei_2d, W_te, b_te, W1, b1, W2, b2, W3, b3, W_ce, b_ce):
    grid = (B // BB,)
    BT = 64
    (x_nei_T,) = pl.pallas_call(
        _transpose_kernel,
        grid=(B // BT,),
        in_specs=[pl.BlockSpec((BT, NEI, OBS * 2), lambda i: (i, 0, 0))],
        out_specs=[
            pl.BlockSpec((BT, OBS * 2, NEI), lambda i: (i, 0, 0)),
        ],
        out_shape=[
            jax.ShapeDtypeStruct((B, OBS * 2, NEI), jnp.float32),
        ],
        compiler_params=pltpu.CompilerParams(
            dimension_semantics=("parallel",)),
    )(x_nei_2d.reshape(B, NEI, OBS * 2))

    t_ids = jnp.arange(OBS)
    W_big = (jnp.zeros((OBS, 2, OBS, D_H), jnp.float32)
             .at[t_ids, :, t_ids, :]
             .set(jnp.broadcast_to(W_te, (OBS, 2, D_H)))
             .reshape(OBS * 2, OBS * D_H))
    b_te2 = b_te.reshape(1, D_H)
    b1_2 = b1.reshape(1, D_H)
    b2_2 = b2.reshape(1, D_H)
    b3_2 = b3.reshape(1, D // 2)
    b_ce2 = b_ce.reshape(1, D // 2)

    def rep(shape):
        return pl.BlockSpec(shape, lambda i: (0,) * len(shape))

    f_re, re_flat = pl.pallas_call(
        _fused_kernel,
        grid=grid,
        in_specs=[
            pl.BlockSpec((BB, OBS, 2), lambda i: (i, 0, 0)),
            pl.BlockSpec((BB, NEI, OBS * 2), lambda i: (i, 0, 0)),
            pl.BlockSpec((BB, OBS * 2, NEI), lambda i: (i, 0, 0)),
            rep((OBS * 2, OBS * D_H)), rep((1, D_H)),
            rep((D_H, D_H)), rep((1, D_H)),
            rep((D_H, D_H)), rep((1, D_H)),
            rep((D_H, D // 2)), rep((1, D // 2)),
            rep((2, D // 2)), rep((1, D // 2)),
        ],
        out_specs=[
            pl.BlockSpec((BB, NEI, OBS * (D // 2)), lambda i: (i, 0, 0)),
            pl.BlockSpec((BB, OBS * PARTITIONS, D), lambda i: (i, 0, 0)),
        ],
        out_shape=[
            jax.ShapeDtypeStruct((B, NEI, OBS * (D // 2)), jnp.float32),
            jax.ShapeDtypeStruct((B, OBS * PARTITIONS, D), jnp.float32),
        ],
        compiler_params=pltpu.CompilerParams(
            dimension_semantics=("parallel",)),
    )(x_ego_2d, x_nei_2d.reshape(B, NEI, OBS * 2), x_nei_T, W_big, b_te2,
      W1, b1_2, W2, b2_2, W3, b3_2, W_ce, b_ce2)

    re_matrix = re_flat.reshape(B, OBS, PARTITIONS, D)
    return (re_matrix, f_re.reshape(B, NEI, OBS, D // 2))


# speedup vs baseline: 1.0515x; 1.0515x over previous
import jax
import jax.numpy as jnp
import numpy as np
from jax.experimental import pallas as pl
from jax.experimental.pallas import tpu as pltpu

PARTITIONS = 8
D_H = 128
D = 128
OBS = 8
B = 1024
NEI = 64

BB = 16


def _transpose_kernel(x_ref, o_ref):
    o_ref[...] = jnp.transpose(x_ref[...], (0, 2, 1))


def _fused_kernel(x_ego_ref, x_nei16_ref, x_nei_T_ref, W_big_ref, b_te_ref,
                  W1_ref, b1_ref, W2_ref, b2_ref, W3_ref, b3_ref,
                  W_ce_ref, b_ce_ref, f_re_ref, re_ref):
    bb = x_ego_ref.shape[0]
    f32 = jnp.float32
    x_ego = x_ego_ref[...]
    ego_last = x_ego[:, OBS - 1:OBS, :]

    x16 = x_nei16_ref[...]
    xw = jnp.dot(x16.reshape(bb * NEI, OBS * 2), W_big_ref[...],
                 preferred_element_type=f32).reshape(bb, NEI, OBS, D_H)
    f_nei = jax.nn.relu(xw - xw[:, :, OBS - 1:OBS, :] + b_te_ref[...])

    xe = jnp.dot(x_ego.reshape(bb, OBS * 2), W_big_ref[...],
                 preferred_element_type=f32).reshape(bb, OBS, D_H)
    f_ego = jax.nn.relu(xe - xe[:, OBS - 1:OBS, :] + b_te_ref[...])
    f_ego = f_ego.reshape(bb, 1, OBS, D_H)

    f = (f_ego * f_nei).reshape(bb * NEI * OBS, D_H)

    h = jax.nn.relu(
        jnp.dot(f, W1_ref[...], preferred_element_type=f32) + b1_ref[...])
    h = jax.nn.relu(
        jnp.dot(h, W2_ref[...], preferred_element_type=f32) + b2_ref[...])
    f_re = jax.nn.relu(
        jnp.dot(h, W3_ref[...], preferred_element_type=f32) + b3_ref[...])
    f_re_ref[...] = f_re.reshape(bb, NEI, OBS, D // 2)

    xT = x_nei_T_ref[...].reshape(bb, OBS, 2, NEI)
    ego_dx = (ego_last[:, :, 0] - x_ego[:, :, 0])[..., None]
    ego_dy = (ego_last[:, :, 1] - x_ego[:, :, 1])[..., None]
    px_t = xT[:, :, 0, :] + ego_dx
    py_t = xT[:, :, 1, :] + ego_dy
    dist_t = jnp.sqrt(px_t * px_t + py_t * py_t)
    ang_t = jnp.arctan2(px_t, py_t) % (2.0 * np.pi)
    idx_t = (ang_t / (2.0 * np.pi / PARTITIONS)).astype(jnp.int32)
    valid = (jnp.abs(px_t + py_t) > 1e-6) & (dist_t > 0.005)
    t_iota = jax.lax.broadcasted_iota(jnp.int32, (1, OBS, 1), 1)
    code_t = jnp.where(valid, idx_t + PARTITIONS * t_iota, -1)

    c_iota = jax.lax.broadcasted_iota(jnp.int32, (1, 1, 1, OBS * PARTITIONS), 3)

    code = jnp.transpose(code_t, (0, 2, 1))
    M = (code[..., None] == c_iota).astype(f32)
    out_re = jax.lax.dot_general(
        M.reshape(bb, NEI * OBS, OBS * PARTITIONS),
        f_re.reshape(bb, NEI * OBS, D // 2),
        (((1,), (1,)), ((0,), (0,))),
        preferred_element_type=f32)

    M_t = (code_t[..., None] == c_iota).astype(f32)
    ones = jnp.ones((bb, 1, OBS, NEI), f32)
    V = jnp.concatenate([dist_t[:, None], ang_t[:, None], ones], axis=1)
    out_geo = jax.lax.dot_general(
        V.reshape(bb, 3, OBS * NEI),
        M_t.reshape(bb, OBS * NEI, OBS * PARTITIONS),
        (((2,), (1,)), ((0,), (0,))),
        preferred_element_type=f32)

    dsum = jnp.transpose(out_geo[:, 0:1, :], (0, 2, 1))
    asum = jnp.transpose(out_geo[:, 1:2, :], (0, 2, 1))
    cnt = jnp.transpose(out_geo[:, 2:3, :], (0, 2, 1))

    inv_n = 1.0 / (cnt + 0.0001)
    re_part = out_re * inv_n
    d_mean = dsum * inv_n
    a_mean = asum * inv_n

    wc0 = W_ce_ref[0:1, :].reshape(1, 1, D // 2)
    wc1 = W_ce_ref[1:2, :].reshape(1, 1, D // 2)
    f_pos = jax.nn.relu(d_mean * wc0 + a_mean * wc1
                        + b_ce_ref[...].reshape(1, 1, D // 2))

    re_ref[...] = jnp.concatenate([re_part, f_pos], axis=-1)


@jax.jit
def kernel(x_ego_2d, x_nei_2d, W_te, b_te, W1, b1, W2, b2, W3, b3, W_ce, b_ce):
    grid = (B // BB,)
    BT = 64
    (x_nei_T,) = pl.pallas_call(
        _transpose_kernel,
        grid=(B // BT,),
        in_specs=[pl.BlockSpec((BT, NEI, OBS * 2), lambda i: (i, 0, 0))],
        out_specs=[
            pl.BlockSpec((BT, OBS * 2, NEI), lambda i: (i, 0, 0)),
        ],
        out_shape=[
            jax.ShapeDtypeStruct((B, OBS * 2, NEI), jnp.float32),
        ],
        compiler_params=pltpu.CompilerParams(
            dimension_semantics=("parallel",)),
    )(x_nei_2d.reshape(B, NEI, OBS * 2))

    t_ids = jnp.arange(OBS)
    W_big = (jnp.zeros((OBS, 2, OBS, D_H), jnp.float32)
             .at[t_ids, :, t_ids, :]
             .set(jnp.broadcast_to(W_te, (OBS, 2, D_H)))
             .reshape(OBS * 2, OBS * D_H))
    b_te2 = b_te.reshape(1, D_H)
    b1_2 = b1.reshape(1, D_H)
    b2_2 = b2.reshape(1, D_H)
    b3_2 = b3.reshape(1, D // 2)
    b_ce2 = b_ce.reshape(1, D // 2)

    def rep(shape):
        return pl.BlockSpec(shape, lambda i: (0,) * len(shape))

    f_re, re_flat = pl.pallas_call(
        _fused_kernel,
        grid=grid,
        in_specs=[
            pl.BlockSpec((BB, OBS, 2), lambda i: (i, 0, 0)),
            pl.BlockSpec((BB, NEI, OBS * 2), lambda i: (i, 0, 0)),
            pl.BlockSpec((BB, OBS * 2, NEI), lambda i: (i, 0, 0)),
            rep((OBS * 2, OBS * D_H)), rep((1, D_H)),
            rep((D_H, D_H)), rep((1, D_H)),
            rep((D_H, D_H)), rep((1, D_H)),
            rep((D_H, D // 2)), rep((1, D // 2)),
            rep((2, D // 2)), rep((1, D // 2)),
        ],
        out_specs=[
            pl.BlockSpec((BB, NEI, OBS, D // 2), lambda i: (i, 0, 0, 0)),
            pl.BlockSpec((BB, OBS * PARTITIONS, D), lambda i: (i, 0, 0)),
        ],
        out_shape=[
            jax.ShapeDtypeStruct((B, NEI, OBS, D // 2), jnp.float32),
            jax.ShapeDtypeStruct((B, OBS * PARTITIONS, D), jnp.float32),
        ],
        compiler_params=pltpu.CompilerParams(
            dimension_semantics=("parallel",)),
    )(x_ego_2d, x_nei_2d.reshape(B, NEI, OBS * 2), x_nei_T, W_big, b_te2,
      W1, b1_2, W2, b2_2, W3, b3_2, W_ce, b_ce2)

    re_matrix = re_flat.reshape(B, OBS, PARTITIONS, D)
    return (re_matrix, f_re)
